# baseline (speedup 1.0000x reference)
import functools

import jax
import jax.numpy as jnp
from jax import lax
from jax.experimental import pallas as pl
from jax.experimental.pallas import tpu as pltpu
from jax.experimental.pallas import tpu_sc as plsc

B, V, P, DV, DP, H = 1024, 200, 256, 64, 32, 512
KA = V * DV
KP = P * DP
BK = 512
NKA = KA // BK
NKP = KP // BK
NEG = -1e8

NW = 32
BPW = B // NW


def _sc_gather_pair(table2, sel):
    @functools.partial(
        pl.kernel,
        out_type=jax.ShapeDtypeStruct((B, 2 * DV), jnp.float32),
        mesh=plsc.VectorSubcoreMesh(core_axis_name="c", subcore_axis_name="s"),
        scratch_types=[
            pltpu.VMEM((BPW,), jnp.int32),
            pltpu.VMEM((BPW, 2 * DV), jnp.float32),
            pltpu.SemaphoreType.DMA,
        ],
    )
    def k(table_hbm, sel_hbm, out_hbm, idx_v, rows_v, sem):
        wid = lax.axis_index("s") * 2 + lax.axis_index("c")
        base = wid * BPW
        pltpu.sync_copy(sel_hbm.at[pl.ds(base, BPW)], idx_v)
        for c in range(BPW // 16):
            off = c * 16
            rows = lax.iota(jnp.int32, 16) + (base + off)
            s = idx_v[pl.ds(off, 16)]
            idx_v[pl.ds(off, 16)] = rows * (V // 2) + lax.shift_right_logical(s, 1)
        pltpu.async_copy(table_hbm.at[idx_v], rows_v, sem).wait()
        pltpu.sync_copy(rows_v, out_hbm.at[pl.ds(base, BPW)])

    return k(table2, sel)


def _mm_body(avm_ref, pm_ref, w1_ref, w1pb_ref, h_ref, p2_ref):
    k = pl.program_id(0)

    @pl.when(k == 0)
    def _():
        h_ref[...] = jnp.zeros_like(h_ref)
        p2_ref[...] = jnp.zeros_like(p2_ref)

    w1 = w1_ref[...].astype(jnp.bfloat16)

    @pl.when(k < NKA)
    def _():
        x = avm_ref[...].reshape(B, BK).astype(jnp.bfloat16)
        h_ref[...] += jnp.dot(x, w1, preferred_element_type=jnp.float32)

    @pl.when(k >= NKA)
    def _():
        x = pm_ref[...].reshape(B, BK).astype(jnp.bfloat16)
        h_ref[...] += jnp.dot(x, w1, preferred_element_type=jnp.float32)
        p2_ref[...] += jnp.dot(x, w1pb_ref[...].astype(jnp.bfloat16),
                               preferred_element_type=jnp.float32)


def _tc_matmul(avm, pm, W1, W1p):
    return pl.pallas_call(
        _mm_body,
        grid=(NKA + NKP,),
        in_specs=[
            pl.BlockSpec((B, BK // DV, DV),
                         lambda k: (0, jnp.minimum(k, NKA - 1), 0)),
            pl.BlockSpec((B, BK // DP, DP),
                         lambda k: (0, jnp.clip(k - NKA, 0, NKP - 1), 0)),
            pl.BlockSpec((BK, H), lambda k: (k, 0)),
            pl.BlockSpec(
                (pl.Element(BK), pl.Element(H)),
                lambda k: (pl.multiple_of(
                    DV + jnp.clip(k - NKA, 0, NKP - 1) * BK, DV), 0)),
        ],
        out_specs=[
            pl.BlockSpec((B, H), lambda k: (0, 0)),
            pl.BlockSpec((B, H), lambda k: (0, 0)),
        ],
        out_shape=[jax.ShapeDtypeStruct((B, H), jnp.float32)] * 2,
        compiler_params=pltpu.CompilerParams(
            dimension_semantics=("arbitrary",)),
    )(avm, pm, W1, W1p)


def _masked_cat(logits, mask, cols, sel):
    ml = jnp.where(mask, NEG, logits)
    m = jnp.max(ml, axis=1, keepdims=True)
    e = jnp.exp(ml - m)
    s = jnp.sum(e, axis=1, keepdims=True)
    lse = jnp.log(s) + m
    logp = ml - lse
    p = e / s
    ent = -jnp.sum(jnp.where(mask, 0.0, p * logp), axis=1, keepdims=True)
    lp = jnp.sum(jnp.where(cols == sel, logp, 0.0), axis=1, keepdims=True)
    return lp, ent


def _head_body(h_ref, p2_ref, pair_ref, w1pa_ref, b1_ref, b1p_ref,
               wl_ref, bl_ref, wc_ref, bc_ref, wlp_ref, blp_ref,
               nvms_ref, selvm_ref, selpm_ref, pmmask_ref,
               lp_ref, ent_ref, cr_ref):
    h = jnp.tanh(h_ref[...] + b1_ref[...])
    vm_logits = jnp.dot(h, wl_ref[...],
                        preferred_element_type=jnp.float32) + bl_ref[...]
    critic = jnp.dot(h, wc_ref[...],
                     preferred_element_type=jnp.float32) + bc_ref[...]
    selvm = selvm_ref[...]
    cols = lax.broadcasted_iota(jnp.int32, (B, V), 1)
    maskv = cols >= nvms_ref[...]
    lpv, entv = _masked_cat(vm_logits, maskv, cols, selvm)

    pair = pair_ref[...]
    vm_sel = jnp.where(lax.rem(selvm, 2) == 0, pair[:, :DV], pair[:, DV:])
    hp = jnp.tanh(p2_ref[...]
                  + jnp.dot(vm_sel, w1pa_ref[...],
                            preferred_element_type=jnp.float32)
                  + b1p_ref[...])
    pm_logits = jnp.dot(hp, wlp_ref[...],
                        preferred_element_type=jnp.float32) + blp_ref[...]
    colsp = lax.broadcasted_iota(jnp.int32, (B, P), 1)
    lpp, entp = _masked_cat(pm_logits, pmmask_ref[...], colsp, selpm_ref[...])

    lp_ref[...] = lpv + lpp
    ent_ref[...] = entv + entp
    cr_ref[...] = critic


def _tc_head(h_pre, p2, pair, W1p, b1r, b1pr, Wl, blr, Wc, bcr, Wlp, blpr,
             nvms, selvm, selpm, pm_mask):
    full = lambda a: pl.BlockSpec(a.shape, lambda i: (0,) * a.ndim)
    args = (h_pre, p2, pair, W1p, b1r, b1pr, Wl, blr, Wc, bcr, Wlp, blpr,
            nvms, selvm, selpm, pm_mask)
    in_specs = [full(a) for a in args]
    in_specs[3] = pl.BlockSpec((DV, H), lambda i: (0, 0))
    return pl.pallas_call(
        _head_body,
        grid=(1,),
        in_specs=in_specs,
        out_specs=[pl.BlockSpec((B, 1), lambda i: (0, 0))] * 3,
        out_shape=[jax.ShapeDtypeStruct((B, 1), jnp.float32)] * 3,
    )(*args)


def kernel(obs_info_pm, obs_info_all_vm, obs_info_num_steps, obs_info_num_vms,
           pm_mask, selected_vm, selected_pm,
           W1, b1, Wl, bl, Wc, bc, W1p, b1p, Wlp, blp):
    table2 = obs_info_all_vm.reshape(B * V // 2, 2 * DV)
    selvm32 = selected_vm.astype(jnp.int32)

    pair = _sc_gather_pair(table2, selvm32)
    h_pre, p2 = _tc_matmul(obs_info_all_vm, obs_info_pm, W1, W1p)

    lp, ent, cr = _tc_head(
        h_pre, p2, pair, W1p, b1.reshape(1, H), b1p.reshape(1, H),
        Wl, bl.reshape(1, V), Wc, bc.reshape(1, 1), Wlp, blp.reshape(1, P),
        obs_info_num_vms.astype(jnp.int32).reshape(B, 1),
        selvm32.reshape(B, 1),
        selected_pm.astype(jnp.int32).reshape(B, 1),
        pm_mask)

    return (selected_vm, selected_pm, lp.reshape(B), ent.reshape(B),
            cr.reshape(B), pm_mask)


# speedup vs baseline: 2.4457x; 2.4457x over previous
import jax
import jax.numpy as jnp
from jax import lax
from jax.experimental import pallas as pl
from jax.experimental.pallas import tpu as pltpu

B, V, P, DV, DP, H = 1024, 200, 256, 64, 32, 512
KA = V * DV
KP = P * DP
BK = 512
NKA = KA // BK
NKP = KP // BK
VPB = BK // DV
NEG = -1e8
TDIMS = (((0,), (0,)), ((), ()))


def _mm_avm_body(xt_ref, selvm_ref, w1_ref, h_ref, vsl_ref):
    k = pl.program_id(0)

    @pl.when(k == 0)
    def _():
        h_ref[...] = jnp.zeros_like(h_ref)
        vsl_ref[...] = jnp.zeros_like(vsl_ref)

    xt = xt_ref[...]
    h_ref[...] += lax.dot_general(
        xt.astype(jnp.bfloat16), w1_ref[...].astype(jnp.bfloat16),
        TDIMS, preferred_element_type=jnp.float32)
    sel = selvm_ref[...]
    acc = vsl_ref[...]
    for i in range(VPB):
        acc += jnp.where(sel == k * VPB + i, xt[i * DV:(i + 1) * DV, :], 0.0)
    vsl_ref[...] = acc


def _tc_avm(xt, selvm_row, W1):
    return pl.pallas_call(
        _mm_avm_body,
        grid=(NKA,),
        in_specs=[
            pl.BlockSpec((BK, B), lambda k: (k, 0)),
            pl.BlockSpec((1, B), lambda k: (0, 0)),
            pl.BlockSpec((BK, H), lambda k: (k, 0)),
        ],
        out_specs=[
            pl.BlockSpec((B, H), lambda k: (0, 0)),
            pl.BlockSpec((DV, B), lambda k: (0, 0)),
        ],
        out_shape=[jax.ShapeDtypeStruct((B, H), jnp.float32),
                   jax.ShapeDtypeStruct((DV, B), jnp.float32)],
        compiler_params=pltpu.CompilerParams(
            dimension_semantics=("arbitrary",)),
    )(xt, selvm_row, W1)


def _mm_pm_body(pm_ref, w1_ref, w1pb_ref, hb_ref, p2_ref):
    j = pl.program_id(0)

    @pl.when(j == 0)
    def _():
        hb_ref[...] = jnp.zeros_like(hb_ref)
        p2_ref[...] = jnp.zeros_like(p2_ref)

    x = pm_ref[...].astype(jnp.bfloat16)
    hb_ref[...] += jnp.dot(x, w1_ref[...].astype(jnp.bfloat16),
                           preferred_element_type=jnp.float32)
    p2_ref[...] += jnp.dot(x, w1pb_ref[...].astype(jnp.bfloat16),
                           preferred_element_type=jnp.float32)


def _tc_pm(pm_flat, W1, W1p):
    return pl.pallas_call(
        _mm_pm_body,
        grid=(NKP,),
        in_specs=[
            pl.BlockSpec((B, BK), lambda j: (0, j)),
            pl.BlockSpec((BK, H), lambda j: (NKA + j, 0)),
            pl.BlockSpec((pl.Element(BK), pl.Element(H)),
                         lambda j: (pl.multiple_of(DV + j * BK, DV), 0)),
        ],
        out_specs=[
            pl.BlockSpec((B, H), lambda j: (0, 0)),
            pl.BlockSpec((B, H), lambda j: (0, 0)),
        ],
        out_shape=[jax.ShapeDtypeStruct((B, H), jnp.float32)] * 2,
        compiler_params=pltpu.CompilerParams(
            dimension_semantics=("arbitrary",)),
    )(pm_flat, W1, W1p)


def _masked_cat(logits, mask, cols, sel):
    ml = jnp.where(mask, NEG, logits)
    m = jnp.max(ml, axis=1, keepdims=True)
    e = jnp.exp(ml - m)
    s = jnp.sum(e, axis=1, keepdims=True)
    lse = jnp.log(s) + m
    logp = ml - lse
    p = e / s
    ent = -jnp.sum(jnp.where(mask, 0.0, p * logp), axis=1, keepdims=True)
    lp = jnp.sum(jnp.where(cols == sel, logp, 0.0), axis=1, keepdims=True)
    return lp, ent


def _head_body(ha_ref, hb_ref, p2_ref, vsl_ref, w1pa_ref, b1_ref, b1p_ref,
               wl_ref, bl_ref, wc_ref, bc_ref, wlp_ref, blp_ref,
               nvms_ref, selvm_ref, selpm_ref, pmmask_ref,
               lp_ref, ent_ref, cr_ref):
    h = jnp.tanh(ha_ref[...] + hb_ref[...] + b1_ref[...])
    vm_logits = jnp.dot(h, wl_ref[...],
                        preferred_element_type=jnp.float32) + bl_ref[...]
    critic = jnp.dot(h, wc_ref[...],
                     preferred_element_type=jnp.float32) + bc_ref[...]
    selvm = selvm_ref[...]
    cols = lax.broadcasted_iota(jnp.int32, (B, V), 1)
    maskv = cols >= nvms_ref[...]
    lpv, entv = _masked_cat(vm_logits, maskv, cols, selvm)

    hp = jnp.tanh(p2_ref[...]
                  + lax.dot_general(vsl_ref[...], w1pa_ref[...], TDIMS,
                                    preferred_element_type=jnp.float32)
                  + b1p_ref[...])
    pm_logits = jnp.dot(hp, wlp_ref[...],
                        preferred_element_type=jnp.float32) + blp_ref[...]
    colsp = lax.broadcasted_iota(jnp.int32, (B, P), 1)
    lpp, entp = _masked_cat(pm_logits, pmmask_ref[...], colsp, selpm_ref[...])

    lp_ref[...] = lpv + lpp
    ent_ref[...] = entv + entp
    cr_ref[...] = critic


def _tc_head(ha, hb, p2, vsl, W1p, b1r, b1pr, Wl, blr, Wc, bcr, Wlp, blpr,
             nvms, selvm, selpm, pm_mask):
    full = lambda a: pl.BlockSpec(a.shape, lambda i: (0,) * a.ndim)
    args = (ha, hb, p2, vsl, W1p, b1r, b1pr, Wl, blr, Wc, bcr, Wlp, blpr,
            nvms, selvm, selpm, pm_mask)
    in_specs = [full(a) for a in args]
    in_specs[4] = pl.BlockSpec((DV, H), lambda i: (0, 0))
    return pl.pallas_call(
        _head_body,
        grid=(1,),
        in_specs=in_specs,
        out_specs=[pl.BlockSpec((B, 1), lambda i: (0, 0))] * 3,
        out_shape=[jax.ShapeDtypeStruct((B, 1), jnp.float32)] * 3,
    )(*args)


def kernel(obs_info_pm, obs_info_all_vm, obs_info_num_steps, obs_info_num_vms,
           pm_mask, selected_vm, selected_pm,
           W1, b1, Wl, bl, Wc, bc, W1p, b1p, Wlp, blp):
    xt = jnp.transpose(obs_info_all_vm, (1, 2, 0)).reshape(KA, B)
    pm_flat = obs_info_pm.reshape(B, KP)
    selvm32 = selected_vm.astype(jnp.int32)

    h_a, vsl = _tc_avm(xt, selvm32.reshape(1, B), W1)
    h_b, p2 = _tc_pm(pm_flat, W1, W1p)

    lp, ent, cr = _tc_head(
        h_a, h_b, p2, vsl, W1p, b1.reshape(1, H), b1p.reshape(1, H),
        Wl, bl.reshape(1, V), Wc, bc.reshape(1, 1), Wlp, blp.reshape(1, P),
        obs_info_num_vms.astype(jnp.int32).reshape(B, 1),
        selvm32.reshape(B, 1),
        selected_pm.astype(jnp.int32).reshape(B, 1),
        pm_mask)

    return (selected_vm, selected_pm, lp.reshape(B), ent.reshape(B),
            cr.reshape(B), pm_mask)


# speedup vs baseline: 2.4499x; 1.0017x over previous
import jax
import jax.numpy as jnp
from jax import lax
from jax.experimental import pallas as pl
from jax.experimental.pallas import tpu as pltpu

B, V, P, DV, DP, H = 1024, 200, 256, 64, 32, 512
KA = V * DV
KP = P * DP
BK = 512
NKA = KA // BK
NKP = KP // BK
NSTEP = NKA + NKP + 1
VPB = BK // DV
NEG = -1e8
TDIMS = (((0,), (0,)), ((), ()))


def _masked_cat(logits, mask, cols, sel):
    ml = jnp.where(mask, NEG, logits)
    m = jnp.max(ml, axis=1, keepdims=True)
    e = jnp.exp(ml - m)
    s = jnp.sum(e, axis=1, keepdims=True)
    lse = jnp.log(s) + m
    logp = ml - lse
    p = e / s
    ent = -jnp.sum(jnp.where(mask, 0.0, p * logp), axis=1, keepdims=True)
    lp = jnp.sum(jnp.where(cols == sel, logp, 0.0), axis=1, keepdims=True)
    return lp, ent


def _body(xt_ref, pm_ref, w1_ref, w1pb_ref, selvmr_ref,
          w1pa_ref, b1_ref, b1p_ref, wl_ref, bl_ref, wc_ref, bc_ref,
          wlp_ref, blp_ref, nvms_ref, selvm_ref, selpm_ref, pmmask_ref,
          lp_ref, ent_ref, cr_ref,
          h_acc, p2_acc, vsl_acc):
    k = pl.program_id(0)

    @pl.when(k == 0)
    def _():
        h_acc[...] = jnp.zeros_like(h_acc)
        p2_acc[...] = jnp.zeros_like(p2_acc)
        vsl_acc[...] = jnp.zeros_like(vsl_acc)

    @pl.when(k < NKA)
    def _():
        xt = xt_ref[...]
        h_acc[...] += lax.dot_general(
            xt.astype(jnp.bfloat16), w1_ref[...].astype(jnp.bfloat16),
            TDIMS, preferred_element_type=jnp.float32)
        sel = selvmr_ref[...]
        acc = vsl_acc[...]
        for i in range(VPB):
            acc += jnp.where(sel == k * VPB + i,
                             xt[i * DV:(i + 1) * DV, :], 0.0)
        vsl_acc[...] = acc

    @pl.when(jnp.logical_and(k >= NKA, k < NKA + NKP))
    def _():
        x = pm_ref[...].astype(jnp.bfloat16)
        h_acc[...] += jnp.dot(x, w1_ref[...].astype(jnp.bfloat16),
                              preferred_element_type=jnp.float32)
        p2_acc[...] += jnp.dot(x, w1pb_ref[...].astype(jnp.bfloat16),
                               preferred_element_type=jnp.float32)

    @pl.when(k == NSTEP - 1)
    def _():
        h = jnp.tanh(h_acc[...] + b1_ref[...])
        vm_logits = jnp.dot(h, wl_ref[...],
                            preferred_element_type=jnp.float32) + bl_ref[...]
        critic = jnp.dot(h, wc_ref[...],
                         preferred_element_type=jnp.float32) + bc_ref[...]
        selvm = selvm_ref[...]
        cols = lax.broadcasted_iota(jnp.int32, (B, V), 1)
        maskv = cols >= nvms_ref[...]
        lpv, entv = _masked_cat(vm_logits, maskv, cols, selvm)

        hp = jnp.tanh(p2_acc[...]
                      + lax.dot_general(vsl_acc[...], w1pa_ref[...], TDIMS,
                                        preferred_element_type=jnp.float32)
                      + b1p_ref[...])
        pm_logits = jnp.dot(hp, wlp_ref[...],
                            preferred_element_type=jnp.float32) + blp_ref[...]
        colsp = lax.broadcasted_iota(jnp.int32, (B, P), 1)
        lpp, entp = _masked_cat(pm_logits, pmmask_ref[...], colsp,
                                selpm_ref[...])

        lp_ref[...] = lpv + lpp
        ent_ref[...] = entv + entp
        cr_ref[...] = critic


def _fused(xt, pm_flat, W1, W1p, selvm_row, b1r, b1pr, Wl, blr, Wc, bcr,
           Wlp, blpr, nvms, selvm, selpm, pm_mask):
    full = lambda a: pl.BlockSpec(a.shape, lambda k: (0,) * a.ndim)
    in_specs = [
        pl.BlockSpec((BK, B), lambda k: (jnp.minimum(k, NKA - 1), 0)),
        pl.BlockSpec((B, BK), lambda k: (0, jnp.clip(k - NKA, 0, NKP - 1))),
        pl.BlockSpec((BK, H), lambda k: (jnp.minimum(k, NKA + NKP - 1), 0)),
        pl.BlockSpec((pl.Element(BK), pl.Element(H)),
                     lambda k: (pl.multiple_of(
                         DV + jnp.clip(k - NKA, 0, NKP - 1) * BK, DV), 0)),
        pl.BlockSpec((1, B), lambda k: (0, 0)),
        pl.BlockSpec((DV, H), lambda k: (0, 0)),
        full(b1r), full(b1pr), full(Wl), full(blr), full(Wc), full(bcr),
        full(Wlp), full(blpr), full(nvms), full(selvm), full(selpm),
        full(pm_mask),
    ]
    return pl.pallas_call(
        _body,
        grid=(NSTEP,),
        in_specs=in_specs,
        out_specs=[pl.BlockSpec((B, 1), lambda k: (0, 0))] * 3,
        out_shape=[jax.ShapeDtypeStruct((B, 1), jnp.float32)] * 3,
        scratch_shapes=[
            pltpu.VMEM((B, H), jnp.float32),
            pltpu.VMEM((B, H), jnp.float32),
            pltpu.VMEM((DV, B), jnp.float32),
        ],
        compiler_params=pltpu.CompilerParams(
            dimension_semantics=("arbitrary",)),
    )(xt, pm_flat, W1, W1p, selvm_row, W1p, b1r, b1pr, Wl, blr, Wc, bcr,
      Wlp, blpr, nvms, selvm, selpm, pm_mask)


def kernel(obs_info_pm, obs_info_all_vm, obs_info_num_steps, obs_info_num_vms,
           pm_mask, selected_vm, selected_pm,
           W1, b1, Wl, bl, Wc, bc, W1p, b1p, Wlp, blp):
    xt = jnp.transpose(obs_info_all_vm, (1, 2, 0)).reshape(KA, B)
    pm_flat = obs_info_pm.reshape(B, KP)
    selvm32 = selected_vm.astype(jnp.int32)

    lp, ent, cr = _fused(
        xt, pm_flat, W1, W1p, selvm32.reshape(1, B),
        b1.reshape(1, H), b1p.reshape(1, H), Wl, bl.reshape(1, V),
        Wc, bc.reshape(1, 1), Wlp, blp.reshape(1, P),
        obs_info_num_vms.astype(jnp.int32).reshape(B, 1),
        selvm32.reshape(B, 1),
        selected_pm.astype(jnp.int32).reshape(B, 1),
        pm_mask)

    return (selected_vm, selected_pm, lp.reshape(B), ent.reshape(B),
            cr.reshape(B), pm_mask)


# speedup vs baseline: 2.7567x; 1.1252x over previous
import jax
import jax.numpy as jnp
from jax import lax
from jax.experimental import pallas as pl
from jax.experimental.pallas import tpu as pltpu

B, V, P, DV, DP, H = 1024, 200, 256, 64, 32, 512
KA = V * DV
KP = P * DP
BK = 512
NKA = KA // BK
NKP = KP // BK
NSTEP = NKA + NKP + 1
VPB = BK // DV
NEG = -1e8
TDIMS = (((0,), (0,)), ((), ()))


def _masked_cat(logits, mask, cols, sel):
    ml = jnp.where(mask, NEG, logits)
    m = jnp.max(ml, axis=1, keepdims=True)
    e = jnp.exp(ml - m)
    s = jnp.sum(e, axis=1, keepdims=True)
    lse = jnp.log(s) + m
    logp = ml - lse
    p = e / s
    ent = -jnp.sum(jnp.where(mask, 0.0, p * logp), axis=1, keepdims=True)
    lp = jnp.sum(jnp.where(cols == sel, logp, 0.0), axis=1, keepdims=True)
    return lp, ent


def _body(xt_ref, pm_ref, w1_ref, w1pb_ref, selvmr_ref,
          w1pa_ref, b1_ref, b1p_ref, wl_ref, bl_ref, wc_ref, bc_ref,
          wlp_ref, blp_ref, nvms_ref, selvm_ref, selpm_ref, pmmask_ref,
          lp_ref, ent_ref, cr_ref,
          h_acc, p2_acc, vsl_acc):
    k = pl.program_id(0)

    @pl.when(k == 0)
    def _():
        h_acc[...] = jnp.zeros_like(h_acc)
        p2_acc[...] = jnp.zeros_like(p2_acc)
        vsl_acc[...] = jnp.zeros_like(vsl_acc)

    @pl.when(k < NKA)
    def _():
        xt = xt_ref[...]
        h_acc[...] += lax.dot_general(
            xt.astype(jnp.bfloat16), w1_ref[...].astype(jnp.bfloat16),
            TDIMS, preferred_element_type=jnp.float32)
        sel = selvmr_ref[...]
        acc = vsl_acc[...]
        for i in range(VPB):
            acc += jnp.where(sel == k * VPB + i,
                             xt[i * DV:(i + 1) * DV, :], 0.0)
        vsl_acc[...] = acc

    @pl.when(jnp.logical_and(k >= NKA, k < NKA + NKP))
    def _():
        x = pm_ref[...]
        h_acc[...] += jnp.dot(x, w1_ref[...].astype(jnp.bfloat16),
                              preferred_element_type=jnp.float32)
        p2_acc[...] += jnp.dot(x, w1pb_ref[...].astype(jnp.bfloat16),
                               preferred_element_type=jnp.float32)

    @pl.when(k == NSTEP - 1)
    def _():
        h = jnp.tanh(h_acc[...] + b1_ref[...])
        vm_logits = jnp.dot(h, wl_ref[...],
                            preferred_element_type=jnp.float32) + bl_ref[...]
        critic = jnp.dot(h, wc_ref[...],
                         preferred_element_type=jnp.float32) + bc_ref[...]
        selvm = selvm_ref[...]
        cols = lax.broadcasted_iota(jnp.int32, (B, V), 1)
        maskv = cols >= nvms_ref[...]
        lpv, entv = _masked_cat(vm_logits, maskv, cols, selvm)

        hp = jnp.tanh(p2_acc[...]
                      + lax.dot_general(vsl_acc[...], w1pa_ref[...], TDIMS,
                                        preferred_element_type=jnp.float32)
                      + b1p_ref[...])
        pm_logits = jnp.dot(hp, wlp_ref[...],
                            preferred_element_type=jnp.float32) + blp_ref[...]
        colsp = lax.broadcasted_iota(jnp.int32, (B, P), 1)
        lpp, entp = _masked_cat(pm_logits, pmmask_ref[...], colsp,
                                selpm_ref[...])

        lp_ref[...] = lpv + lpp
        ent_ref[...] = entv + entp
        cr_ref[...] = critic


def _fused(xt, pm_flat, W1, W1p, selvm_row, b1r, b1pr, Wl, blr, Wc, bcr,
           Wlp, blpr, nvms, selvm, selpm, pm_mask):
    full = lambda a: pl.BlockSpec(a.shape, lambda k: (0,) * a.ndim)
    in_specs = [
        pl.BlockSpec((BK, B), lambda k: (jnp.minimum(k, NKA - 1), 0)),
        pl.BlockSpec((B, BK), lambda k: (0, jnp.clip(k - NKA, 0, NKP - 1))),
        pl.BlockSpec((BK, H), lambda k: (jnp.minimum(k, NKA + NKP - 1), 0)),
        pl.BlockSpec((pl.Element(BK), pl.Element(H)),
                     lambda k: (pl.multiple_of(
                         DV + jnp.clip(k - NKA, 0, NKP - 1) * BK, DV), 0)),
        pl.BlockSpec((1, B), lambda k: (0, 0)),
        pl.BlockSpec((DV, H), lambda k: (0, 0)),
        full(b1r), full(b1pr), full(Wl), full(blr), full(Wc), full(bcr),
        full(Wlp), full(blpr), full(nvms), full(selvm), full(selpm),
        full(pm_mask),
    ]
    return pl.pallas_call(
        _body,
        grid=(NSTEP,),
        in_specs=in_specs,
        out_specs=[pl.BlockSpec((B, 1), lambda k: (0, 0))] * 3,
        out_shape=[jax.ShapeDtypeStruct((B, 1), jnp.float32)] * 3,
        scratch_shapes=[
            pltpu.VMEM((B, H), jnp.float32),
            pltpu.VMEM((B, H), jnp.float32),
            pltpu.VMEM((DV, B), jnp.float32),
        ],
        compiler_params=pltpu.CompilerParams(
            dimension_semantics=("arbitrary",)),
    )(xt, pm_flat, W1, W1p, selvm_row, W1p, b1r, b1pr, Wl, blr, Wc, bcr,
      Wlp, blpr, nvms, selvm, selpm, pm_mask)


def kernel(obs_info_pm, obs_info_all_vm, obs_info_num_steps, obs_info_num_vms,
           pm_mask, selected_vm, selected_pm,
           W1, b1, Wl, bl, Wc, bc, W1p, b1p, Wlp, blp):
    xt = jnp.transpose(obs_info_all_vm, (1, 2, 0)).reshape(KA, B)
    pm_flat = obs_info_pm.astype(jnp.bfloat16).reshape(B, KP)
    selvm32 = selected_vm.astype(jnp.int32)

    lp, ent, cr = _fused(
        xt, pm_flat, W1, W1p, selvm32.reshape(1, B),
        b1.reshape(1, H), b1p.reshape(1, H), Wl, bl.reshape(1, V),
        Wc, bc.reshape(1, 1), Wlp, blp.reshape(1, P),
        obs_info_num_vms.astype(jnp.int32).reshape(B, 1),
        selvm32.reshape(B, 1),
        selected_pm.astype(jnp.int32).reshape(B, 1),
        pm_mask)

    return (selected_vm, selected_pm, lp.reshape(B), ent.reshape(B),
            cr.reshape(B), pm_mask)
